# Initial kernel scaffold; baseline (speedup 1.0000x reference)
import functools

import jax
import jax.numpy as jnp
from jax.experimental import pallas as pl
from jax.experimental.pallas import tpu as pltpu

N_TOK = 2048
D_MODEL = 768
D_FF = 3072
N_EXP = 8
TOP_K = 2

TB = 1024
NB = N_TOK // TB


def _gelu_tanh(x):
    c = 0.7978845608028654
    return 0.5 * x * (1.0 + jnp.tanh(c * (x + 0.044715 * x * x * x)))


def _moe_body(x_ref, wg_ref, w1_ref, b1_ref, w2_ref, b2_ref,
              out_ref, loss_ref, gates_ref, cnt_ref, psum_ref):
    i = pl.program_id(0)
    e = pl.program_id(1)
    lane = jax.lax.broadcasted_iota(jnp.int32, (TB, N_EXP), 1)

    @pl.when(e == 0)
    def _router():
        xb = x_ref[...]
        logits = jnp.dot(xb, wg_ref[...], preferred_element_type=jnp.float32)
        m = jnp.max(logits, axis=1, keepdims=True)
        p = jnp.exp(logits - m)
        probs = p / jnp.sum(p, axis=1, keepdims=True)
        m1 = jnp.max(probs, axis=1, keepdims=True)
        i1 = jnp.min(jnp.where(probs == m1, lane, N_EXP), axis=1, keepdims=True)
        sel1 = lane == i1
        probs2 = jnp.where(sel1, -jnp.inf, probs)
        m2 = jnp.max(probs2, axis=1, keepdims=True)
        i2 = jnp.min(jnp.where(probs2 == m2, lane, N_EXP), axis=1, keepdims=True)
        sel2 = lane == i2
        denom = m1 + m2
        gates_ref[...] = (jnp.where(sel1, m1 / denom, 0.0)
                          + jnp.where(sel2, m2 / denom, 0.0))
        cnt_part = jnp.sum((sel1 | sel2).astype(jnp.float32), axis=0,
                           keepdims=True)
        p_part = jnp.sum(probs, axis=0, keepdims=True)

        @pl.when(i == 0)
        def _init():
            cnt_ref[...] = cnt_part
            psum_ref[...] = p_part

        @pl.when(i > 0)
        def _acc():
            cnt_ref[...] += cnt_part
            psum_ref[...] += p_part

    xb16 = x_ref[...].astype(jnp.bfloat16)
    h = jnp.dot(xb16, w1_ref[0], preferred_element_type=jnp.float32)
    h = _gelu_tanh(h + b1_ref[...])
    y = jnp.dot(h.astype(jnp.bfloat16), w2_ref[0],
                preferred_element_type=jnp.float32) + b2_ref[...]
    g = jnp.sum(jnp.where(lane == e, gates_ref[...], 0.0), axis=1,
                keepdims=True)

    @pl.when(e == 0)
    def _out0():
        out_ref[...] = g * y

    @pl.when(e > 0)
    def _outn():
        out_ref[...] += g * y

    @pl.when((i == NB - 1) & (e == N_EXP - 1))
    def _loss():
        frac = cnt_ref[...] / N_TOK
        mean_p = psum_ref[...] / N_TOK
        loss_ref[...] = (N_EXP * jnp.sum(frac * mean_p)).reshape(1, 1)


@jax.jit
def _moe(x, Wg, W1b, b1, W2b, b2):
    out, loss = pl.pallas_call(
        _moe_body,
        grid=(NB, N_EXP),
        in_specs=[
            pl.BlockSpec((TB, D_MODEL), lambda i, e: (i, 0)),
            pl.BlockSpec((D_MODEL, N_EXP), lambda i, e: (0, 0)),
            pl.BlockSpec((1, D_MODEL, D_FF), lambda i, e: (e, 0, 0)),
            pl.BlockSpec((1, D_FF), lambda i, e: (e, 0)),
            pl.BlockSpec((1, D_FF, D_MODEL), lambda i, e: (e, 0, 0)),
            pl.BlockSpec((1, D_MODEL), lambda i, e: (e, 0)),
        ],
        out_specs=[
            pl.BlockSpec((TB, D_MODEL), lambda i, e: (i, 0)),
            pl.BlockSpec((1, 1), lambda i, e: (0, 0)),
        ],
        out_shape=[
            jax.ShapeDtypeStruct((N_TOK, D_MODEL), jnp.float32),
            jax.ShapeDtypeStruct((1, 1), jnp.float32),
        ],
        scratch_shapes=[
            pltpu.VMEM((TB, N_EXP), jnp.float32),
            pltpu.VMEM((1, N_EXP), jnp.float32),
            pltpu.VMEM((1, N_EXP), jnp.float32),
        ],
        compiler_params=pltpu.CompilerParams(
            dimension_semantics=("arbitrary", "arbitrary")),
    )(x, Wg, W1b, b1, W2b, b2)
    return out, loss


def kernel(x, Wg, W1, b1, W2, b2):
    out, loss = _moe(x, Wg, W1.astype(jnp.bfloat16), b1,
                     W2.astype(jnp.bfloat16), b2)
    return out, loss.reshape(())


# speedup vs baseline: 1.1063x; 1.1063x over previous
import functools

import jax
import jax.numpy as jnp
from jax.experimental import pallas as pl
from jax.experimental.pallas import tpu as pltpu

N_TOK = 2048
D_MODEL = 768
D_FF = 3072
N_EXP = 8
TOP_K = 2

TB = 1024
NB = N_TOK // TB


def _gelu_tanh(x):
    c = 0.7978845608028654
    return 0.5 * x * (1.0 + jnp.tanh(c * (x + 0.044715 * x * x * x)))


def _moe_body(x_ref, wg_ref, w1_ref, b1_ref, w2_ref, b2_ref,
              out_ref, loss_ref, gates_ref, cnt_ref, psum_ref):
    i = pl.program_id(0)
    e = pl.program_id(1)
    lane = jax.lax.broadcasted_iota(jnp.int32, (TB, N_EXP), 1)

    @pl.when(e == 0)
    def _router():
        xb = x_ref[...]
        logits = jnp.dot(xb, wg_ref[...], preferred_element_type=jnp.float32)
        m = jnp.max(logits, axis=1, keepdims=True)
        p = jnp.exp(logits - m)
        probs = p / jnp.sum(p, axis=1, keepdims=True)
        m1 = jnp.max(probs, axis=1, keepdims=True)
        i1 = jnp.min(jnp.where(probs == m1, lane, N_EXP), axis=1, keepdims=True)
        sel1 = lane == i1
        probs2 = jnp.where(sel1, -jnp.inf, probs)
        m2 = jnp.max(probs2, axis=1, keepdims=True)
        i2 = jnp.min(jnp.where(probs2 == m2, lane, N_EXP), axis=1, keepdims=True)
        sel2 = lane == i2
        denom = m1 + m2
        gates_ref[...] = (jnp.where(sel1, m1 / denom, 0.0)
                          + jnp.where(sel2, m2 / denom, 0.0))
        cnt_part = jnp.sum((sel1 | sel2).astype(jnp.float32), axis=0,
                           keepdims=True)
        p_part = jnp.sum(probs, axis=0, keepdims=True)

        @pl.when(i == 0)
        def _init():
            cnt_ref[...] = cnt_part
            psum_ref[...] = p_part

        @pl.when(i > 0)
        def _acc():
            cnt_ref[...] += cnt_part
            psum_ref[...] += p_part

    xb16 = x_ref[...].astype(jnp.bfloat16)
    h = jnp.dot(xb16, w1_ref[0], preferred_element_type=jnp.float32)
    h = _gelu_tanh(h + b1_ref[0])
    y = jnp.dot(h.astype(jnp.bfloat16), w2_ref[0],
                preferred_element_type=jnp.float32) + b2_ref[0]
    g = jnp.sum(jnp.where(lane == e, gates_ref[...], 0.0), axis=1,
                keepdims=True)

    @pl.when(e == 0)
    def _out0():
        out_ref[...] = g * y

    @pl.when(e > 0)
    def _outn():
        out_ref[...] += g * y

    @pl.when((i == NB - 1) & (e == N_EXP - 1))
    def _loss():
        frac = cnt_ref[...] / N_TOK
        mean_p = psum_ref[...] / N_TOK
        loss_ref[...] = (N_EXP * jnp.sum(frac * mean_p)).reshape(1, 1)


@jax.jit
def _moe(x, Wg, W1b, b1, W2b, b2):
    out, loss = pl.pallas_call(
        _moe_body,
        grid=(NB, N_EXP),
        in_specs=[
            pl.BlockSpec((TB, D_MODEL), lambda i, e: (i, 0)),
            pl.BlockSpec((D_MODEL, N_EXP), lambda i, e: (0, 0)),
            pl.BlockSpec((1, D_MODEL, D_FF), lambda i, e: (e, 0, 0)),
            pl.BlockSpec((1, 1, D_FF), lambda i, e: (e, 0, 0)),
            pl.BlockSpec((1, D_FF, D_MODEL), lambda i, e: (e, 0, 0)),
            pl.BlockSpec((1, 1, D_MODEL), lambda i, e: (e, 0, 0)),
        ],
        out_specs=[
            pl.BlockSpec((TB, D_MODEL), lambda i, e: (i, 0)),
            pl.BlockSpec((1, 1), lambda i, e: (0, 0)),
        ],
        out_shape=[
            jax.ShapeDtypeStruct((N_TOK, D_MODEL), jnp.float32),
            jax.ShapeDtypeStruct((1, 1), jnp.float32),
        ],
        scratch_shapes=[
            pltpu.VMEM((TB, N_EXP), jnp.float32),
            pltpu.VMEM((1, N_EXP), jnp.float32),
            pltpu.VMEM((1, N_EXP), jnp.float32),
        ],
        compiler_params=pltpu.CompilerParams(
            dimension_semantics=("arbitrary", "arbitrary")),
    )(x, Wg, W1b, b1, W2b, b2)
    return out, loss


def kernel(x, Wg, W1, b1, W2, b2):
    out, loss = _moe(x, Wg, W1.astype(jnp.bfloat16),
                     b1.reshape(N_EXP, 1, D_FF),
                     W2.astype(jnp.bfloat16),
                     b2.reshape(N_EXP, 1, D_MODEL))
    return out, loss.reshape(())
